# baseline (device time: 86492 ns/iter reference)
import jax
import jax.numpy as jnp
from jax import lax
from jax.experimental import pallas as pl
from jax.experimental.pallas import tpu as pltpu

N_DEV = 16
DEPTH = 3


def kernel(x, w_mat):
    m_loc, k = x.shape
    _, n = w_mat.shape
    n_loc = n // N_DEV
    m = m_loc * N_DEV

    def body(x_ref, w_hbm, out_ref,
             xb_ref, w_ring, asm, send_buf, scal_buf,
             w_sems, send_sems, recv_sems, ssend_sems, srecv_sems):
        me = lax.axis_index("i")

        def w_copy(t):
            d = (me + 1 + t) % N_DEV
            return pltpu.make_async_copy(
                w_hbm.at[:, pl.ds(d * n_loc, n_loc)],
                w_ring.at[t % DEPTH],
                w_sems.at[t % DEPTH],
            )

        for t in range(DEPTH):
            w_copy(t).start()

        xb_ref[...] = x_ref[...].astype(jnp.bfloat16)

        amax = jnp.float32(0.0)
        for t in range(N_DEV):
            d = (me + 1 + t) % N_DEV
            w_copy(t).wait()
            wb = w_ring[t % DEPTH].astype(jnp.bfloat16)
            blk = jnp.dot(xb_ref[...], wb,
                          preferred_element_type=jnp.float32)
            bb = blk.astype(jnp.bfloat16)
            amax = jnp.maximum(amax, jnp.max(jnp.abs(bb.astype(jnp.float32))))
            if t < N_DEV - 1:
                send_buf[d] = bb
                rdma = pltpu.make_async_remote_copy(
                    src_ref=send_buf.at[d],
                    dst_ref=asm.at[me],
                    send_sem=send_sems.at[d],
                    recv_sem=recv_sems.at[me],
                    device_id=(d,),
                    device_id_type=pl.DeviceIdType.MESH,
                )
                rdma.start()
            else:
                asm[me] = bb
            if t + DEPTH < N_DEV:
                w_copy(t + DEPTH).start()

        scal_buf[me] = jnp.full((8, 128), amax, jnp.float32)
        for s in range(1, N_DEV):
            dd = (me + s) % N_DEV
            pltpu.make_async_remote_copy(
                src_ref=scal_buf.at[me],
                dst_ref=scal_buf.at[me],
                send_sem=ssend_sems.at[dd],
                recv_sem=srecv_sems.at[me],
                device_id=(dd,),
                device_id_type=pl.DeviceIdType.MESH,
            ).start()

        for s in range(1, N_DEV):
            src = (me + s) % N_DEV
            pltpu.make_async_remote_copy(
                src_ref=scal_buf.at[src],
                dst_ref=scal_buf.at[src],
                send_sem=ssend_sems.at[src],
                recv_sem=srecv_sems.at[src],
                device_id=(src,),
                device_id_type=pl.DeviceIdType.MESH,
            ).wait_recv()
        amax_g = jnp.max(scal_buf[...])
        scale = amax_g / 448.0
        inv = 448.0 / amax_g

        def quant_store(s_idx, blk_bf16):
            y = blk_bf16.astype(jnp.float32) * inv
            q = jnp.clip(y, -448.0, 448.0)
            q = q.astype(jnp.float8_e4m3fn).astype(jnp.float32) * scale
            out_ref[pl.ds(s_idx * m_loc, m_loc), :] = q

        quant_store(me, asm[me])
        for j in range(1, N_DEV):
            src = (me - j) % N_DEV
            pltpu.make_async_remote_copy(
                src_ref=asm.at[src],
                dst_ref=asm.at[src],
                send_sem=send_sems.at[src],
                recv_sem=recv_sems.at[src],
                device_id=(src,),
                device_id_type=pl.DeviceIdType.MESH,
            ).wait_recv()
            quant_store(src, asm[src])

        for s in range(1, N_DEV):
            dd = (me + s) % N_DEV
            pltpu.make_async_remote_copy(
                src_ref=send_buf.at[dd],
                dst_ref=asm.at[me],
                send_sem=send_sems.at[dd],
                recv_sem=recv_sems.at[me],
                device_id=(dd,),
                device_id_type=pl.DeviceIdType.MESH,
            ).wait_send()
            pltpu.make_async_remote_copy(
                src_ref=scal_buf.at[me],
                dst_ref=scal_buf.at[me],
                send_sem=ssend_sems.at[dd],
                recv_sem=srecv_sems.at[me],
                device_id=(dd,),
                device_id_type=pl.DeviceIdType.MESH,
            ).wait_send()

    return pl.pallas_call(
        body,
        in_specs=[
            pl.BlockSpec(memory_space=pltpu.VMEM),
            pl.BlockSpec(memory_space=pl.ANY),
        ],
        out_specs=pl.BlockSpec(memory_space=pltpu.VMEM),
        out_shape=jax.ShapeDtypeStruct((m, n_loc), jnp.float32),
        compiler_params=pltpu.CompilerParams(
            vmem_limit_bytes=60 * 1024 * 1024,
        ),
        scratch_shapes=[
            pltpu.VMEM((m_loc, k), jnp.bfloat16),
            pltpu.VMEM((DEPTH, k, n_loc), jnp.float32),
            pltpu.VMEM((N_DEV, m_loc, n_loc), jnp.bfloat16),
            pltpu.VMEM((N_DEV, m_loc, n_loc), jnp.bfloat16),
            pltpu.VMEM((N_DEV, 8, 128), jnp.float32),
            pltpu.SemaphoreType.DMA((DEPTH,)),
            pltpu.SemaphoreType.DMA((N_DEV,)),
            pltpu.SemaphoreType.DMA((N_DEV,)),
            pltpu.SemaphoreType.DMA((N_DEV,)),
            pltpu.SemaphoreType.DMA((N_DEV,)),
        ],
    )(x, w_mat)


# device time: 83898 ns/iter; 1.0309x vs baseline; 1.0309x over previous
import jax
import jax.numpy as jnp
from jax import lax
from jax.experimental import pallas as pl
from jax.experimental.pallas import tpu as pltpu

N_DEV = 16
EARLY = 7


def kernel(x, w_mat):
    m_loc, k = x.shape
    _, n = w_mat.shape
    n_loc = n // N_DEV
    m = m_loc * N_DEV

    my = lax.axis_index("i")
    perm = (my + 1 + jnp.arange(N_DEV, dtype=jnp.int32)) % N_DEV

    def body(perm_ref, x_ref, w_ref, out_ref,
             xb_ref, asm, f8_asm, send_buf, f8_send, scal_buf, amax_ref,
             send_sems, recv_sems, f8send_sems, f8recv_sems,
             ssend_sems, srecv_sems):
        t = pl.program_id(0)
        me = lax.axis_index("i")
        d = perm_ref[t]

        @pl.when(t == 0)
        def _():
            xb_ref[...] = x_ref[...].astype(jnp.bfloat16)

        wb = w_ref[...].astype(jnp.bfloat16)
        blk = jnp.dot(xb_ref[...], wb, preferred_element_type=jnp.float32)
        bb = blk.astype(jnp.bfloat16)
        bmax = jnp.max(jnp.abs(bb.astype(jnp.float32)))

        @pl.when(t == 0)
        def _():
            amax_ref[0] = bmax

        @pl.when(t != 0)
        def _():
            amax_ref[0] = jnp.maximum(amax_ref[0], bmax)

        @pl.when(t < N_DEV - 1)
        def _():
            send_buf[d] = bb

        @pl.when(t < EARLY)
        def _():
            pltpu.make_async_remote_copy(
                src_ref=send_buf.at[d],
                dst_ref=asm.at[me],
                send_sem=send_sems.at[d],
                recv_sem=recv_sems.at[me],
                device_id=(d,),
                device_id_type=pl.DeviceIdType.MESH,
            ).start()

        @pl.when(t == N_DEV - 1)
        def _():
            asm[me] = bb

            scal_buf[me] = jnp.full((8, 128), amax_ref[0], jnp.float32)
            for s in range(1, N_DEV):
                dd = (me + s) % N_DEV
                pltpu.make_async_remote_copy(
                    src_ref=scal_buf.at[me],
                    dst_ref=scal_buf.at[me],
                    send_sem=ssend_sems.at[dd],
                    recv_sem=srecv_sems.at[me],
                    device_id=(dd,),
                    device_id_type=pl.DeviceIdType.MESH,
                ).start()
            for s in range(1, N_DEV):
                src = (me + s) % N_DEV
                pltpu.make_async_remote_copy(
                    src_ref=scal_buf.at[src],
                    dst_ref=scal_buf.at[src],
                    send_sem=ssend_sems.at[src],
                    recv_sem=srecv_sems.at[src],
                    device_id=(src,),
                    device_id_type=pl.DeviceIdType.MESH,
                ).wait_recv()
            amax_g = jnp.max(scal_buf[...])
            scale = amax_g / 448.0
            inv = 448.0 / amax_g

            def to_f8(blk_bf16):
                y = blk_bf16.astype(jnp.float32) * inv
                q = jnp.clip(y, -448.0, 448.0)
                return q.astype(jnp.float8_e4m3fn)

            for s in range(EARLY + 1, N_DEV):
                dd = (me + s) % N_DEV
                f8_send[dd] = to_f8(send_buf[dd])
                pltpu.make_async_remote_copy(
                    src_ref=f8_send.at[dd],
                    dst_ref=f8_asm.at[me],
                    send_sem=f8send_sems.at[dd],
                    recv_sem=f8recv_sems.at[me],
                    device_id=(dd,),
                    device_id_type=pl.DeviceIdType.MESH,
                ).start()

            def quant_store(s_idx, blk_bf16):
                q = to_f8(blk_bf16)
                out_ref[pl.ds(s_idx * m_loc, m_loc), :] = (
                    q.astype(jnp.float32) * scale)

            quant_store(me, asm[me])
            for j in range(1, EARLY + 1):
                src = (me - j) % N_DEV
                pltpu.make_async_remote_copy(
                    src_ref=asm.at[src],
                    dst_ref=asm.at[src],
                    send_sem=send_sems.at[src],
                    recv_sem=recv_sems.at[src],
                    device_id=(src,),
                    device_id_type=pl.DeviceIdType.MESH,
                ).wait_recv()
                quant_store(src, asm[src])
            for j in range(EARLY + 1, N_DEV):
                src = (me - j) % N_DEV
                pltpu.make_async_remote_copy(
                    src_ref=f8_asm.at[src],
                    dst_ref=f8_asm.at[src],
                    send_sem=f8send_sems.at[src],
                    recv_sem=f8recv_sems.at[src],
                    device_id=(src,),
                    device_id_type=pl.DeviceIdType.MESH,
                ).wait_recv()
                out_ref[pl.ds(src * m_loc, m_loc), :] = (
                    f8_asm[src].astype(jnp.float32) * scale)

            for s in range(1, N_DEV):
                dd = (me + s) % N_DEV
                if s <= EARLY:
                    pltpu.make_async_remote_copy(
                        src_ref=send_buf.at[dd],
                        dst_ref=asm.at[me],
                        send_sem=send_sems.at[dd],
                        recv_sem=recv_sems.at[me],
                        device_id=(dd,),
                        device_id_type=pl.DeviceIdType.MESH,
                    ).wait_send()
                else:
                    pltpu.make_async_remote_copy(
                        src_ref=f8_send.at[dd],
                        dst_ref=f8_asm.at[me],
                        send_sem=f8send_sems.at[dd],
                        recv_sem=f8recv_sems.at[me],
                        device_id=(dd,),
                        device_id_type=pl.DeviceIdType.MESH,
                    ).wait_send()
                pltpu.make_async_remote_copy(
                    src_ref=scal_buf.at[me],
                    dst_ref=scal_buf.at[me],
                    send_sem=ssend_sems.at[dd],
                    recv_sem=srecv_sems.at[me],
                    device_id=(dd,),
                    device_id_type=pl.DeviceIdType.MESH,
                ).wait_send()

    grid_spec = pltpu.PrefetchScalarGridSpec(
        num_scalar_prefetch=1,
        grid=(N_DEV,),
        in_specs=[
            pl.BlockSpec((m_loc, k), lambda t, p: (0, 0)),
            pl.BlockSpec((k, n_loc), lambda t, p: (0, p[t])),
        ],
        out_specs=pl.BlockSpec((m, n_loc), lambda t, p: (0, 0)),
        scratch_shapes=[
            pltpu.VMEM((m_loc, k), jnp.bfloat16),
            pltpu.VMEM((N_DEV, m_loc, n_loc), jnp.bfloat16),
            pltpu.VMEM((N_DEV, m_loc, n_loc), jnp.float8_e4m3fn),
            pltpu.VMEM((N_DEV, m_loc, n_loc), jnp.bfloat16),
            pltpu.VMEM((N_DEV, m_loc, n_loc), jnp.float8_e4m3fn),
            pltpu.VMEM((N_DEV, 8, 128), jnp.float32),
            pltpu.SMEM((1,), jnp.float32),
            pltpu.SemaphoreType.DMA((N_DEV,)),
            pltpu.SemaphoreType.DMA((N_DEV,)),
            pltpu.SemaphoreType.DMA((N_DEV,)),
            pltpu.SemaphoreType.DMA((N_DEV,)),
            pltpu.SemaphoreType.DMA((N_DEV,)),
            pltpu.SemaphoreType.DMA((N_DEV,)),
        ],
    )
    return pl.pallas_call(
        body,
        grid_spec=grid_spec,
        out_shape=jax.ShapeDtypeStruct((m, n_loc), jnp.float32),
        compiler_params=pltpu.CompilerParams(
            dimension_semantics=("arbitrary",),
            vmem_limit_bytes=60 * 1024 * 1024,
        ),
    )(perm, x, w_mat)
